# baseline (speedup 1.0000x reference)
import jax
import jax.numpy as jnp
from jax.experimental import pallas as pl
from jax.experimental.pallas import tpu as pltpu

N = 10000
NFEAT = 128
NHID = 128
NCLASS = 64

BI = 640
GRID = (N + BI - 1) // BI
N_PAD = GRID * BI


def _gcn_kernel(x_ref, adj_ref, w1_ref, b1_ref, w2_ref, b2_ref,
                out_ref, s2_ref):
    g = pl.program_id(0)

    @pl.when(g < GRID)
    def _phase1():
        ax = jnp.dot(adj_ref[...], x_ref[...],
                     preferred_element_type=jnp.float32)
        h = jnp.maximum(
            jnp.dot(ax, w1_ref[...], preferred_element_type=jnp.float32)
            + b1_ref[...][None, :], 0.0)
        s2_ref[pl.ds(g * BI, BI), :] = jnp.dot(
            h, w2_ref[...], preferred_element_type=jnp.float32)

    @pl.when(g >= GRID)
    def _phase2():
        o = jnp.dot(adj_ref[...], s2_ref[pl.ds(0, N), :],
                    preferred_element_type=jnp.float32) + b2_ref[...][None, :]
        m = jnp.max(o, axis=1, keepdims=True)
        shifted = o - m
        lse = jnp.log(jnp.sum(jnp.exp(shifted), axis=1, keepdims=True))
        out_ref[...] = shifted - lse


def _adj_index(g):
    return (jnp.where(g >= GRID, g - GRID, g), 0)


def _out_index(g):
    return (jnp.maximum(g - GRID, 0), 0)


@jax.jit
def kernel(x, adj, W1, b1, W2, b2):
    out = pl.pallas_call(
        _gcn_kernel,
        grid=(2 * GRID,),
        in_specs=[
            pl.BlockSpec((N, NFEAT), lambda g: (0, 0)),
            pl.BlockSpec((BI, N), _adj_index),
            pl.BlockSpec((NFEAT, NHID), lambda g: (0, 0)),
            pl.BlockSpec((NHID,), lambda g: (0,)),
            pl.BlockSpec((NHID, NCLASS), lambda g: (0, 0)),
            pl.BlockSpec((NCLASS,), lambda g: (0,)),
        ],
        out_specs=pl.BlockSpec((BI, NCLASS), _out_index),
        out_shape=jax.ShapeDtypeStruct((N, NCLASS), jnp.float32),
        scratch_shapes=[
            pltpu.VMEM((N_PAD, NCLASS), jnp.float32),
        ],
        compiler_params=pltpu.CompilerParams(
            dimension_semantics=("arbitrary",),
            vmem_limit_bytes=64 * 1024 * 1024,
        ),
    )(x, adj, W1, b1, W2, b2)

    return out


# speedup vs baseline: 1.0226x; 1.0226x over previous
import jax
import jax.numpy as jnp
from jax.experimental import pallas as pl
from jax.experimental.pallas import tpu as pltpu

N = 10000
NFEAT = 128
NHID = 128
NCLASS = 64

BI = 400
GRID = N // BI


def _gcn_kernel(x_ref, adj_ref, w1_ref, b1_ref, w2_ref, b2_ref,
                out_ref, s2_ref):
    g = pl.program_id(0)

    @pl.when(g < GRID)
    def _phase1():
        ax = jnp.dot(adj_ref[...], x_ref[...],
                     preferred_element_type=jnp.float32)
        h = jnp.maximum(
            jnp.dot(ax, w1_ref[...], preferred_element_type=jnp.float32)
            + b1_ref[...][None, :], 0.0)
        s2_ref[pl.ds(g * BI, BI), :] = jnp.dot(
            h, w2_ref[...], preferred_element_type=jnp.float32)

    @pl.when(g >= GRID)
    def _phase2():
        o = jnp.dot(adj_ref[...], s2_ref[...],
                    preferred_element_type=jnp.float32) + b2_ref[...][None, :]
        m = jnp.max(o, axis=1, keepdims=True)
        shifted = o - m
        lse = jnp.log(jnp.sum(jnp.exp(shifted), axis=1, keepdims=True))
        out_ref[...] = shifted - lse


def _adj_index(g):
    return (jnp.where(g >= GRID, g - GRID, g), 0)


def _out_index(g):
    return (jnp.maximum(g - GRID, 0), 0)


@jax.jit
def kernel(x, adj, W1, b1, W2, b2):
    out = pl.pallas_call(
        _gcn_kernel,
        grid=(2 * GRID,),
        in_specs=[
            pl.BlockSpec((N, NFEAT), lambda g: (0, 0)),
            pl.BlockSpec((BI, N), _adj_index),
            pl.BlockSpec((NFEAT, NHID), lambda g: (0, 0)),
            pl.BlockSpec((NHID,), lambda g: (0,)),
            pl.BlockSpec((NHID, NCLASS), lambda g: (0, 0)),
            pl.BlockSpec((NCLASS,), lambda g: (0,)),
        ],
        out_specs=pl.BlockSpec((BI, NCLASS), _out_index),
        out_shape=jax.ShapeDtypeStruct((N, NCLASS), jnp.float32),
        scratch_shapes=[
            pltpu.VMEM((N, NCLASS), jnp.float32),
        ],
        compiler_params=pltpu.CompilerParams(
            dimension_semantics=("arbitrary",),
        ),
    )(x, adj, W1, b1, W2, b2)

    return out
